# baseline (speedup 1.0000x reference)
import functools

import jax
import jax.numpy as jnp
from jax import lax
from jax.experimental import pallas as pl
from jax.experimental.pallas import tpu as pltpu
from jax.experimental.pallas import tpu_sc as plsc

N_NODES = 10000
N_EDGES = 320000
D_FEAT = 128
D_EDGE = 16
EDGE_FILTERS = 32
NODE_FILTERS = 64
CLASSES = 40
CLASSES_PAD = 64

N_PAD = 10240

NC = 2
NS = 16
NW = NC * NS
FB = 8
ER = NW // FB
EDGES_PER_RANGE = N_EDGES // ER
CHUNK = 128
NCHUNKS = EDGES_PER_RANGE // CHUNK
ACC_WORDS = N_PAD * FB


def _edge_mlp_body(eft_ref, w1t_ref, b1_ref, w2t_ref, b2_ref, out_ref):
    h = jnp.maximum(
        jnp.dot(w1t_ref[...], eft_ref[...], preferred_element_type=jnp.float32)
        + b1_ref[...], 0.0)
    z = jnp.dot(w2t_ref[...], h, preferred_element_type=jnp.float32) \
        + b2_ref[...]
    out_ref[...] = jax.nn.sigmoid(z)[None]


def _matmul_body(x_ref, w_ref, out_ref):
    out_ref[...] = jnp.dot(x_ref[...], w_ref[...],
                           preferred_element_type=jnp.float32)


def _mid_body(p_ref, bg1_ref, w_ref, out_ref):
    s = p_ref[0] + p_ref[1] + p_ref[2] + p_ref[3]
    h = jnp.maximum(s + bg1_ref[...], 0.0)
    out_ref[...] = jnp.dot(h, w_ref[...], preferred_element_type=jnp.float32)


def _final_body(q_ref, bg2_ref, out_ref):
    logits = (q_ref[0] + q_ref[1] + q_ref[2] + q_ref[3]
              + bg2_ref[...])
    col = lax.broadcasted_iota(jnp.int32, logits.shape, 1)
    valid = col < CLASSES
    masked = jnp.where(valid, logits, -jnp.inf)
    m = jnp.max(masked, axis=1, keepdims=True)
    ex = jnp.where(valid, jnp.exp(logits - m), 0.0)
    lse = jnp.log(jnp.sum(ex, axis=1, keepdims=True)) + m
    out_ref[...] = lse - logits


def _lane_sel(v16, idx):
    return lax.gather(
        v16, idx[:, None],
        dimension_numbers=lax.GatherDimensionNumbers(
            offset_dims=(), collapsed_slice_dims=(0,), start_index_map=(0,)),
        slice_sizes=(1,),
        mode=lax.GatherScatterMode.PROMISE_IN_BOUNDS)


def _make_aggregate():
    mesh = plsc.VectorSubcoreMesh(core_axis_name="c", subcore_axis_name="s",
                                  num_cores=NC, num_subcores=NS)

    @functools.partial(
        pl.kernel,
        out_type=jax.ShapeDtypeStruct((NW, ACC_WORDS), jnp.float32),
        mesh=mesh,
        compiler_params=pltpu.CompilerParams(use_tc_tiling_on_sc=False,
                                             needs_layout_passes=False),
        scratch_types=[
            pltpu.VMEM((4, CHUNK), jnp.int32),
            pltpu.VMEM((4, CHUNK), jnp.int32),
            pltpu.VMEM((4, CHUNK), jnp.float32),
            pltpu.VMEM((4, CHUNK, FB), jnp.float32),
            pltpu.VMEM((ACC_WORDS,), jnp.float32),
            pltpu.SemaphoreType.DMA,
            pltpu.SemaphoreType.DMA,
            pltpu.SemaphoreType.DMA,
            pltpu.SemaphoreType.DMA,
        ],
    )
    def agg(table_hbm, col_hbm, row_hbm, e_hbm, zero_hbm, out_hbm,
            colb, ridxb, eb, rin, acc, isem0, isem1, gsem0, gsem1):
        isem = (isem0, isem1)
        gsem = (gsem0, gsem1)
        cid = lax.axis_index("c")
        sid = lax.axis_index("s")
        wid = cid * NS + sid
        fb = wid // ER
        er = lax.rem(wid, ER)
        base = er * EDGES_PER_RANGE
        i16 = lax.iota(jnp.int32, 16)
        off8 = lax.rem(i16, 8)
        half = i16 >> 3

        pltpu.sync_copy(zero_hbm, acc)

        srcs = (col_hbm, row_hbm, e_hbm)
        bufs = (colb, ridxb, eb)

        def coff(c):
            return base + lax.rem(c, NCHUNKS) * CHUNK

        def fire_idx(c, slot, sem):
            for s_hbm, buf in zip(srcs, bufs):
                pltpu.async_copy(s_hbm.at[pl.ds(coff(c), CHUNK)],
                                 buf.at[slot], sem)

        def wait_idx(c, slot, sem):
            for s_hbm, buf in zip(srcs, bufs):
                pltpu.make_async_copy(s_hbm.at[pl.ds(coff(c), CHUNK)],
                                      buf.at[slot], sem).wait()

        def fire_gather(slot, sem):
            pltpu.async_copy(table_hbm.at[fb].at[colb.at[slot]],
                             rin.at[slot], sem)

        def wait_gather(slot, sem):
            pltpu.make_async_copy(table_hbm.at[fb].at[colb.at[slot]],
                                  rin.at[slot], sem).wait()

        def process(u):
            eb_u = eb.at[u]
            rb_u = ridxb.at[u]
            rin_u = rin.at[u]
            for g in range(CHUNK // 16):
                sl16 = pl.ds(g * 16, 16)
                ev16 = eb_u[sl16]
                addr16 = rb_u[sl16] << 3
                for p in range(8):
                    pat = half + (g * 16 + 2 * p)
                    evp = _lane_sel(ev16, lax.rem(pat, 16))
                    ap = _lane_sel(addr16, lax.rem(pat, 16)) + off8
                    v = plsc.load_gather(rin_u, [pat, off8])
                    plsc.addupdate_scatter(acc, [ap], v * evp)

        def body(j, u, fire=True):
            if fire:
                wait_idx(j + 1, (u + 1) % 4, isem[(u + 1) % 2])
                fire_gather((u + 1) % 4, gsem[(u + 1) % 2])
            wait_gather(u, gsem[u % 2])
            process(u)
            if fire:
                fire_idx(j + 3, (u + 3) % 4, isem[(u + 3) % 2])

        fire_idx(0, 0, isem[0])
        wait_idx(0, 0, isem[0])
        fire_idx(1, 1, isem[1])
        fire_idx(2, 2, isem[0])
        fire_gather(0, gsem[0])

        def quad(jj, carry):
            for u in range(4):
                body(4 * jj + u, u)
            return carry

        lax.fori_loop(0, NCHUNKS // 4, quad, 0)
        body(NCHUNKS - 1, 0, fire=False)

        wait_idx(NCHUNKS, 1, isem[1])
        wait_idx(NCHUNKS + 1, 2, isem[0])
        pltpu.sync_copy(acc, out_hbm.at[wid])

    return agg


_agg = _make_aggregate()


def _split_table(t):
    return t.reshape(N_PAD, FB, 8).transpose(1, 0, 2)


def _merge_partials(o):
    return o.reshape(FB, ER, N_PAD, 8).transpose(1, 2, 0, 3) \
            .reshape(ER, N_PAD, FB * 8)


def kernel(x, edges, edge_fs, W1, b1, W2, b2, Wg1, bg1, Wg2, bg2):
    edges = edges.astype(jnp.int32)
    row = edges[0]
    col = edges[1]

    BE = 32000
    NB = N_EDGES // BE
    eft = edge_fs.T
    e2d = pl.pallas_call(
        _edge_mlp_body,
        grid=(NB,),
        in_specs=[
            pl.BlockSpec((D_EDGE, BE), lambda i: (0, i)),
            pl.BlockSpec((EDGE_FILTERS, D_EDGE), lambda i: (0, 0)),
            pl.BlockSpec((EDGE_FILTERS, 1), lambda i: (0, 0)),
            pl.BlockSpec((1, EDGE_FILTERS), lambda i: (0, 0)),
            pl.BlockSpec((1, 1), lambda i: (0, 0)),
        ],
        out_specs=pl.BlockSpec((1, 1, BE), lambda i: (i, 0, 0)),
        out_shape=jax.ShapeDtypeStruct((NB, 1, BE), jnp.float32),
    )(eft, W1.T, b1.reshape(-1, 1), W2.T, b2.reshape(1, 1))
    e = e2d.reshape(N_EDGES)

    xp = jnp.pad(x, ((0, N_PAD - N_NODES), (0, 0)))
    xw = pl.pallas_call(
        _matmul_body,
        out_shape=jax.ShapeDtypeStruct((N_PAD, NODE_FILTERS), jnp.float32),
    )(xp, Wg1)

    zeros_acc = jnp.zeros((ACC_WORDS,), jnp.float32)

    o1 = _agg(_split_table(xw), col, row, e, zeros_acc)
    p1 = _merge_partials(o1)

    Wg2p = jnp.pad(Wg2, ((0, 0), (0, CLASSES_PAD - CLASSES)))
    hw = pl.pallas_call(
        _mid_body,
        out_shape=jax.ShapeDtypeStruct((N_PAD, CLASSES_PAD), jnp.float32),
    )(p1, bg1.reshape(1, -1), Wg2p)

    o2 = _agg(_split_table(hw), col, row, e, zeros_acc)
    p2 = _merge_partials(o2)

    bg2p = jnp.pad(bg2, (0, CLASSES_PAD - CLASSES))
    outp = pl.pallas_call(
        _final_body,
        out_shape=jax.ShapeDtypeStruct((N_PAD, CLASSES_PAD), jnp.float32),
    )(p2, bg2p.reshape(1, -1))
    return outp[:N_NODES, :CLASSES]


# speedup vs baseline: 3.2542x; 3.2542x over previous
import functools

import jax
import jax.numpy as jnp
from jax import lax
from jax.experimental import pallas as pl
from jax.experimental.pallas import tpu as pltpu
from jax.experimental.pallas import tpu_sc as plsc

N_NODES = 10000
N_EDGES = 320000
D_FEAT = 128
D_EDGE = 16
EDGE_FILTERS = 32
NODE_FILTERS = 64
CLASSES = 40
CLASSES_PAD = 64

N_PAD = 10240

NC = 2
NS = 16
NW = NC * NS
EDGES_PER_WORKER = N_EDGES // NW
CHUNK = 80
NCHUNKS = EDGES_PER_WORKER // CHUNK
ROWS_PER_TILE = N_PAD // NS


def _edge_mlp_body(eft_ref, w1t_ref, b1_ref, w2t_ref, b2_ref, out_ref):
    h = jnp.maximum(
        jnp.dot(w1t_ref[...], eft_ref[...], preferred_element_type=jnp.float32)
        + b1_ref[...], 0.0)
    z = jnp.dot(w2t_ref[...], h, preferred_element_type=jnp.float32) \
        + b2_ref[...]
    out_ref[...] = jax.nn.sigmoid(z)[None]


def _matmul_body(x_ref, w_ref, out_ref):
    out_ref[...] = jnp.dot(x_ref[...], w_ref[...],
                           preferred_element_type=jnp.float32
                           ).astype(jnp.bfloat16)


def _mid_body(p_ref, bg1_ref, w_ref, out_ref):
    h = jnp.maximum(p_ref[0].astype(jnp.float32) + p_ref[1].astype(jnp.float32)
                    + bg1_ref[...], 0.0)
    out_ref[...] = jnp.dot(h, w_ref[...], preferred_element_type=jnp.float32
                           ).astype(jnp.bfloat16)


def _final_body(q_ref, bg2_ref, out_ref):
    logits = (q_ref[0].astype(jnp.float32) + q_ref[1].astype(jnp.float32)
              + bg2_ref[...])
    col = lax.broadcasted_iota(jnp.int32, logits.shape, 1)
    valid = col < CLASSES
    masked = jnp.where(valid, logits, -jnp.inf)
    m = jnp.max(masked, axis=1, keepdims=True)
    ex = jnp.where(valid, jnp.exp(logits - m), 0.0)
    lse = jnp.log(jnp.sum(ex, axis=1, keepdims=True)) + m
    out_ref[...] = lse - logits


def _lane_bcast(v16, i):
    bidx = jnp.full((16,), i, jnp.int32)
    return lax.gather(
        v16, bidx[:, None],
        dimension_numbers=lax.GatherDimensionNumbers(
            offset_dims=(), collapsed_slice_dims=(0,), start_index_map=(0,)),
        slice_sizes=(1,),
        mode=lax.GatherScatterMode.PROMISE_IN_BOUNDS)


def _make_aggregate(d_model):
    assert d_model % 32 == 0
    mesh = plsc.VectorSubcoreMesh(core_axis_name="c", subcore_axis_name="s",
                                  num_cores=NC, num_subcores=NS)

    @functools.partial(
        pl.kernel,
        out_type=jax.ShapeDtypeStruct((NC, N_PAD, d_model), jnp.bfloat16),
        mesh=mesh,
        compiler_params=pltpu.CompilerParams(use_tc_tiling_on_sc=False,
                                            needs_layout_passes=False),
        scratch_types=[
            pltpu.VMEM((4, CHUNK), jnp.int32),
            pltpu.VMEM((4, CHUNK), jnp.int32),
            pltpu.VMEM((4, CHUNK), jnp.float32),
            pltpu.VMEM((4, CHUNK), jnp.int32),
            pltpu.VMEM((4, CHUNK, d_model), jnp.bfloat16),
            pltpu.VMEM((4, CHUNK, d_model), jnp.bfloat16),
            pltpu.VMEM_SHARED((N_PAD, d_model), jnp.bfloat16),
            pltpu.SemaphoreType.DMA,
            pltpu.SemaphoreType.DMA,
            pltpu.SemaphoreType.DMA,
            pltpu.SemaphoreType.DMA,
            pltpu.SemaphoreType.DMA,
            pltpu.SemaphoreType.DMA,
            pltpu.SemaphoreType.DMA,
            pltpu.SemaphoreType.DMA,
        ],
    )
    def agg(table_hbm, col_hbm, row_hbm, e_hbm, zero_hbm, out_hbm,
            colb, ridxb, eb, sridx, rows_in, rows_out, acc,
            isem0, isem1, gsem0, gsem1, ssem0, ssem1, ssem2, ssem3):
        isem = (isem0, isem1)
        gsem = (gsem0, gsem1)
        ssem = (ssem0, ssem1, ssem2, ssem3)
        cid = lax.axis_index("c")
        sid = lax.axis_index("s")
        wid = cid * NS + sid
        r0 = sid * ROWS_PER_TILE
        pltpu.sync_copy(zero_hbm.at[pl.ds(r0, ROWS_PER_TILE)],
                        acc.at[pl.ds(r0, ROWS_PER_TILE)])
        plsc.subcore_barrier()

        base = wid * EDGES_PER_WORKER
        srcs = (col_hbm, row_hbm, e_hbm)
        bufs = (colb, ridxb, eb)

        def coff(c):
            return base + lax.rem(c, NCHUNKS) * CHUNK

        def fire_idx(c, slot, sem):
            for s_hbm, buf in zip(srcs, bufs):
                pltpu.async_copy(s_hbm.at[pl.ds(coff(c), CHUNK)],
                                 buf.at[slot], sem)

        def wait_idx(c, slot, sem):
            for s_hbm, buf in zip(srcs, bufs):
                pltpu.make_async_copy(s_hbm.at[pl.ds(coff(c), CHUNK)],
                                      buf.at[slot], sem).wait()

        def scale(u):
            eb_u = eb.at[u]
            rin = rows_in.at[u]
            rout = rows_out.at[u]
            sr = sridx.at[u]
            rb = ridxb.at[u]
            for g in range(CHUNK // 16):
                sl16 = pl.ds(g * 16, 16)
                sr[sl16] = rb[sl16]
                ev16 = eb_u[sl16]
                for i in range(16):
                    k = g * 16 + i
                    ev = _lane_bcast(ev16, i)
                    evb = plsc.pack(ev, ev, format=plsc.PackFormat.INTERLEAVED)
                    for d0 in range(d_model // 32):
                        sl = pl.ds(d0 * 32, 32)
                        rout[k, sl] = rin[k, sl] * evb

        def body(j, u, fire=True):
            if fire:
                wait_idx(j + 1, (u + 1) % 4, isem[(u + 1) % 2])
                pltpu.async_copy(table_hbm.at[colb.at[(u + 1) % 4]],
                                 rows_in.at[(u + 1) % 4], gsem[(u + 1) % 2])
            pltpu.make_async_copy(table_hbm.at[colb.at[u]],
                                  rows_in.at[u], gsem[u % 2]).wait()
            pltpu.make_async_copy(table_hbm.at[sridx.at[u]],
                                  rows_out.at[u], ssem[u]).wait()
            scale(u)
            pltpu.async_copy(rows_out.at[u], acc.at[sridx.at[u]],
                             ssem[u], add=True)
            if fire:
                fire_idx(j + 3, (u + 3) % 4, isem[(u + 3) % 2])

        for u in range(4):
            sr = sridx.at[u]
            for g in range(CHUNK // 16):
                sr[pl.ds(g * 16, 16)] = jnp.zeros((16,), jnp.int32)
        fire_idx(0, 0, isem[0])
        wait_idx(0, 0, isem[0])
        fire_idx(1, 1, isem[1])
        fire_idx(2, 2, isem[0])
        pltpu.async_copy(table_hbm.at[colb.at[0]], rows_in.at[0], gsem[0])
        for u in range(4):
            pltpu.async_copy(table_hbm.at[sridx.at[u]], rows_out.at[u],
                             ssem[u])

        def quad(jj, carry):
            for u in range(4):
                body(4 * jj + u, u)
            return carry

        lax.fori_loop(0, NCHUNKS // 4, quad, 0)
        body(NCHUNKS - 1, 0, fire=False)

        wait_idx(NCHUNKS, 1, isem[1])
        wait_idx(NCHUNKS + 1, 2, isem[0])
        for u in range(4):
            pltpu.make_async_copy(rows_out.at[u], acc.at[sridx.at[u]],
                                  ssem[u]).wait()
        plsc.subcore_barrier()
        pltpu.sync_copy(acc.at[pl.ds(r0, ROWS_PER_TILE)],
                        out_hbm.at[cid, pl.ds(r0, ROWS_PER_TILE)])

    return agg


_agg64 = _make_aggregate(NODE_FILTERS)
_agg48 = _make_aggregate(CLASSES_PAD)


def kernel(x, edges, edge_fs, W1, b1, W2, b2, Wg1, bg1, Wg2, bg2):
    edges = edges.astype(jnp.int32)
    row = edges[0]
    col = edges[1]

    BE = 32000
    NB = N_EDGES // BE
    eft = edge_fs.T
    e2d = pl.pallas_call(
        _edge_mlp_body,
        grid=(NB,),
        in_specs=[
            pl.BlockSpec((D_EDGE, BE), lambda i: (0, i)),
            pl.BlockSpec((EDGE_FILTERS, D_EDGE), lambda i: (0, 0)),
            pl.BlockSpec((EDGE_FILTERS, 1), lambda i: (0, 0)),
            pl.BlockSpec((1, EDGE_FILTERS), lambda i: (0, 0)),
            pl.BlockSpec((1, 1), lambda i: (0, 0)),
        ],
        out_specs=pl.BlockSpec((1, 1, BE), lambda i: (i, 0, 0)),
        out_shape=jax.ShapeDtypeStruct((NB, 1, BE), jnp.float32),
    )(eft, W1.T, b1.reshape(-1, 1), W2.T, b2.reshape(1, 1))
    e = e2d.reshape(N_EDGES)

    xp = jnp.pad(x, ((0, N_PAD - N_NODES), (0, 0)))
    xw = pl.pallas_call(
        _matmul_body,
        out_shape=jax.ShapeDtypeStruct((N_PAD, NODE_FILTERS), jnp.bfloat16),
    )(xp, Wg1)

    zeros64 = jnp.zeros((N_PAD, NODE_FILTERS), jnp.bfloat16)
    p1 = _agg64(xw, col, row, e, zeros64)

    Wg2p = jnp.pad(Wg2, ((0, 0), (0, CLASSES_PAD - CLASSES)))
    hw = pl.pallas_call(
        _mid_body,
        out_shape=jax.ShapeDtypeStruct((N_PAD, CLASSES_PAD), jnp.bfloat16),
    )(p1, bg1.reshape(1, -1), Wg2p)

    zeros48 = jnp.zeros((N_PAD, CLASSES_PAD), jnp.bfloat16)
    p2 = _agg48(hw, col, row, e, zeros48)

    bg2p = jnp.pad(bg2, (0, CLASSES_PAD - CLASSES))
    outp = pl.pallas_call(
        _final_body,
        out_shape=jax.ShapeDtypeStruct((N_PAD, CLASSES_PAD), jnp.float32),
    )(p2, bg2p.reshape(1, -1))
    return outp[:N_NODES, :CLASSES]


# speedup vs baseline: 3.4562x; 1.0621x over previous
import functools

import jax
import jax.numpy as jnp
from jax import lax
from jax.experimental import pallas as pl
from jax.experimental.pallas import tpu as pltpu
from jax.experimental.pallas import tpu_sc as plsc

N_NODES = 10000
N_EDGES = 320000
D_FEAT = 128
D_EDGE = 16
EDGE_FILTERS = 32
NODE_FILTERS = 64
CLASSES = 40
CLASSES_PAD = 40

N_PAD = 10240

NC = 2
NS = 16
NW = NC * NS
EDGES_PER_WORKER = N_EDGES // NW
CHUNK = 80
NCHUNKS = EDGES_PER_WORKER // CHUNK
ROWS_PER_TILE = N_PAD // NS


def _edge_mlp_body(eft_ref, w1t_ref, b1_ref, w2t_ref, b2_ref, out_ref):
    h = jnp.maximum(
        jnp.dot(w1t_ref[...], eft_ref[...], preferred_element_type=jnp.float32)
        + b1_ref[...], 0.0)
    z = jnp.dot(w2t_ref[...], h, preferred_element_type=jnp.float32) \
        + b2_ref[...]
    out_ref[...] = jax.nn.sigmoid(z)[None]


def _matmul_body(x_ref, w_ref, out_ref):
    xw = jnp.dot(x_ref[...], w_ref[...],
                 preferred_element_type=jnp.float32).astype(jnp.bfloat16)
    out_ref[pl.ds(0, N_NODES), :] = xw
    out_ref[pl.ds(N_NODES, N_PAD - N_NODES), :] = jnp.zeros(
        (N_PAD - N_NODES, NODE_FILTERS), jnp.bfloat16)


def _mid_body(p_ref, bg1_ref, w_ref, out_ref):
    h = jnp.maximum(p_ref[0].astype(jnp.float32) + p_ref[1].astype(jnp.float32)
                    + bg1_ref[...], 0.0)
    out_ref[...] = jnp.dot(h, w_ref[...], preferred_element_type=jnp.float32
                           ).astype(jnp.bfloat16)


def _final_body(q_ref, bg2_ref, out_ref):
    logits = (q_ref[0].astype(jnp.float32) + q_ref[1].astype(jnp.float32)
              + bg2_ref[...])
    col = lax.broadcasted_iota(jnp.int32, logits.shape, 1)
    valid = col < CLASSES
    masked = jnp.where(valid, logits, -jnp.inf)
    m = jnp.max(masked, axis=1, keepdims=True)
    ex = jnp.where(valid, jnp.exp(logits - m), 0.0)
    lse = jnp.log(jnp.sum(ex, axis=1, keepdims=True)) + m
    res = lse - logits
    out_ref[...] = res[:N_NODES, :CLASSES]


def _lane_bcast(v16, i):
    bidx = jnp.full((16,), i, jnp.int32)
    return lax.gather(
        v16, bidx[:, None],
        dimension_numbers=lax.GatherDimensionNumbers(
            offset_dims=(), collapsed_slice_dims=(0,), start_index_map=(0,)),
        slice_sizes=(1,),
        mode=lax.GatherScatterMode.PROMISE_IN_BOUNDS)


def _slice_starts(d_model):
    starts = list(range(0, d_model - 31, 32))
    if d_model % 32:
        starts.append(d_model - 32)
    return starts


def _make_aggregate(d_model):
    assert d_model % 8 == 0 and d_model >= 32
    mesh = plsc.VectorSubcoreMesh(core_axis_name="c", subcore_axis_name="s",
                                  num_cores=NC, num_subcores=NS)

    @functools.partial(
        pl.kernel,
        out_type=jax.ShapeDtypeStruct((NC, N_PAD, d_model), jnp.bfloat16),
        mesh=mesh,
        compiler_params=pltpu.CompilerParams(use_tc_tiling_on_sc=False,
                                            needs_layout_passes=False),
        scratch_types=[
            pltpu.VMEM((4, CHUNK), jnp.int32),
            pltpu.VMEM((4, CHUNK), jnp.int32),
            pltpu.VMEM((4, CHUNK), jnp.float32),
            pltpu.VMEM((4, CHUNK), jnp.int32),
            pltpu.VMEM((4, CHUNK, d_model), jnp.bfloat16),
            pltpu.VMEM((4, CHUNK, d_model), jnp.bfloat16),
            pltpu.VMEM_SHARED((N_PAD, d_model), jnp.bfloat16),
            pltpu.SemaphoreType.DMA,
            pltpu.SemaphoreType.DMA,
            pltpu.SemaphoreType.DMA,
            pltpu.SemaphoreType.DMA,
            pltpu.SemaphoreType.DMA,
            pltpu.SemaphoreType.DMA,
            pltpu.SemaphoreType.DMA,
            pltpu.SemaphoreType.DMA,
        ],
    )
    def agg(table_hbm, col_hbm, row_hbm, e_hbm, zero_hbm, out_hbm,
            colb, ridxb, eb, sridx, rows_in, rows_out, acc,
            isem0, isem1, gsem0, gsem1, ssem0, ssem1, ssem2, ssem3):
        isem = (isem0, isem1)
        gsem = (gsem0, gsem1)
        ssem = (ssem0, ssem1, ssem2, ssem3)
        cid = lax.axis_index("c")
        sid = lax.axis_index("s")
        wid = cid * NS + sid
        r0 = sid * ROWS_PER_TILE
        base = wid * EDGES_PER_WORKER
        srcs = (col_hbm, row_hbm, e_hbm)
        bufs = (colb, ridxb, eb)

        def coff(c):
            return base + lax.rem(c, NCHUNKS) * CHUNK

        def fire_idx(c, slot, sem):
            for s_hbm, buf in zip(srcs, bufs):
                pltpu.async_copy(s_hbm.at[pl.ds(coff(c), CHUNK)],
                                 buf.at[slot], sem)

        def wait_idx(c, slot, sem):
            for s_hbm, buf in zip(srcs, bufs):
                pltpu.make_async_copy(s_hbm.at[pl.ds(coff(c), CHUNK)],
                                      buf.at[slot], sem).wait()

        def scale(u):
            eb_u = eb.at[u]
            rin = rows_in.at[u]
            rout = rows_out.at[u]
            sr = sridx.at[u]
            rb = ridxb.at[u]
            for g in range(CHUNK // 16):
                sl16 = pl.ds(g * 16, 16)
                sr[sl16] = rb[sl16]
                ev16 = eb_u[sl16]
                for i in range(16):
                    k = g * 16 + i
                    ev = _lane_bcast(ev16, i)
                    evb = plsc.pack(ev, ev, format=plsc.PackFormat.INTERLEAVED)
                    for st in _slice_starts(d_model):
                        sl = pl.ds(st, 32)
                        rout[k, sl] = rin[k, sl] * evb

        def body(j, u, fire=True):
            if fire:
                wait_idx(j + 1, (u + 1) % 4, isem[(u + 1) % 2])
                pltpu.async_copy(table_hbm.at[colb.at[(u + 1) % 4]],
                                 rows_in.at[(u + 1) % 4], gsem[(u + 1) % 2])
            pltpu.make_async_copy(table_hbm.at[colb.at[u]],
                                  rows_in.at[u], gsem[u % 2]).wait()
            pltpu.make_async_copy(table_hbm.at[sridx.at[u]],
                                  rows_out.at[u], ssem[u]).wait()
            scale(u)
            pltpu.async_copy(rows_out.at[u], acc.at[sridx.at[u]],
                             ssem[u], add=True)
            if fire:
                fire_idx(j + 3, (u + 3) % 4, isem[(u + 3) % 2])

        for u in range(4):
            sr = sridx.at[u]
            for g in range(CHUNK // 16):
                sr[pl.ds(g * 16, 16)] = jnp.zeros((16,), jnp.int32)
        fire_idx(0, 0, isem[0])
        wait_idx(0, 0, isem[0])
        fire_idx(1, 1, isem[1])
        fire_idx(2, 2, isem[0])
        pltpu.async_copy(table_hbm.at[colb.at[0]], rows_in.at[0], gsem[0])
        pltpu.sync_copy(zero_hbm.at[pl.ds(r0, ROWS_PER_TILE)],
                        acc.at[pl.ds(r0, ROWS_PER_TILE)])
        plsc.subcore_barrier()
        for u in range(4):
            pltpu.async_copy(table_hbm.at[sridx.at[u]], rows_out.at[u],
                             ssem[u])

        def quad(jj, carry):
            for u in range(4):
                body(4 * jj + u, u)
            return carry

        lax.fori_loop(0, NCHUNKS // 4, quad, 0)
        body(NCHUNKS - 1, 0, fire=False)

        wait_idx(NCHUNKS, 1, isem[1])
        wait_idx(NCHUNKS + 1, 2, isem[0])
        for u in range(4):
            pltpu.make_async_copy(rows_out.at[u], acc.at[sridx.at[u]],
                                  ssem[u]).wait()
        plsc.subcore_barrier()
        pltpu.sync_copy(acc.at[pl.ds(r0, ROWS_PER_TILE)],
                        out_hbm.at[cid, pl.ds(r0, ROWS_PER_TILE)])

    return agg


_agg64 = _make_aggregate(NODE_FILTERS)
_agg48 = _make_aggregate(CLASSES_PAD)


def kernel(x, edges, edge_fs, W1, b1, W2, b2, Wg1, bg1, Wg2, bg2):
    edges = edges.astype(jnp.int32)
    row = edges[0]
    col = edges[1]

    BE = 32000
    NB = N_EDGES // BE
    eft = edge_fs.T
    e2d = pl.pallas_call(
        _edge_mlp_body,
        grid=(NB,),
        in_specs=[
            pl.BlockSpec((D_EDGE, BE), lambda i: (0, i)),
            pl.BlockSpec((EDGE_FILTERS, D_EDGE), lambda i: (0, 0)),
            pl.BlockSpec((EDGE_FILTERS, 1), lambda i: (0, 0)),
            pl.BlockSpec((1, EDGE_FILTERS), lambda i: (0, 0)),
            pl.BlockSpec((1, 1), lambda i: (0, 0)),
        ],
        out_specs=pl.BlockSpec((1, 1, BE), lambda i: (i, 0, 0)),
        out_shape=jax.ShapeDtypeStruct((NB, 1, BE), jnp.float32),
    )(eft, W1.T, b1.reshape(-1, 1), W2.T, b2.reshape(1, 1))
    e = e2d.reshape(N_EDGES)

    xw = pl.pallas_call(
        _matmul_body,
        out_shape=jax.ShapeDtypeStruct((N_PAD, NODE_FILTERS), jnp.bfloat16),
    )(x, Wg1)

    zeros64 = jnp.zeros((N_PAD, NODE_FILTERS), jnp.bfloat16)
    p1 = _agg64(xw, col, row, e, zeros64)

    Wg2p = jnp.pad(Wg2, ((0, 0), (0, CLASSES_PAD - CLASSES)))
    hw = pl.pallas_call(
        _mid_body,
        out_shape=jax.ShapeDtypeStruct((N_PAD, CLASSES_PAD), jnp.bfloat16),
    )(p1, bg1.reshape(1, -1), Wg2p)

    zeros48 = jnp.zeros((N_PAD, CLASSES_PAD), jnp.bfloat16)
    p2 = _agg48(hw, col, row, e, zeros48)

    bg2p = jnp.pad(bg2, (0, CLASSES_PAD - CLASSES))
    outp = pl.pallas_call(
        _final_body,
        out_shape=jax.ShapeDtypeStruct((N_NODES, CLASSES), jnp.float32),
    )(p2, bg2p.reshape(1, -1))
    return outp


# speedup vs baseline: 3.4899x; 1.0098x over previous
import functools

import jax
import jax.numpy as jnp
from jax import lax
from jax.experimental import pallas as pl
from jax.experimental.pallas import tpu as pltpu
from jax.experimental.pallas import tpu_sc as plsc

N_NODES = 10000
N_EDGES = 320000
D_FEAT = 128
D_EDGE = 16
EDGE_FILTERS = 32
NODE_FILTERS = 64
CLASSES = 40
CLASSES_PAD = 48

N_PAD = 10240

NC = 2
NS = 16
NW = NC * NS
EDGES_PER_WORKER = N_EDGES // NW
CHUNK = 80
NCHUNKS = EDGES_PER_WORKER // CHUNK
ROWS_PER_TILE = N_PAD // NS


def _edge_mlp_body(eft_ref, w1t_ref, b1_ref, w2t_ref, b2_ref, x_ref, wg1_ref,
                   out_ref, xw_ref):
    h = jnp.maximum(
        jnp.dot(w1t_ref[...], eft_ref[...], preferred_element_type=jnp.float32)
        + b1_ref[...], 0.0)
    z = jnp.dot(w2t_ref[...], h, preferred_element_type=jnp.float32) \
        + b2_ref[...]
    out_ref[...] = jax.nn.sigmoid(z)[None]

    @pl.when(pl.program_id(0) == 0)
    def _():
        xw = jnp.dot(x_ref[...], wg1_ref[...],
                     preferred_element_type=jnp.float32).astype(jnp.bfloat16)
        xw_ref[pl.ds(0, N_NODES), :] = xw
        xw_ref[pl.ds(N_NODES, N_PAD - N_NODES), :] = jnp.zeros(
            (N_PAD - N_NODES, NODE_FILTERS), jnp.bfloat16)


def _mid_body(p_ref, bg1_ref, w_ref, out_ref):
    h = jnp.maximum(p_ref[0].astype(jnp.float32) + p_ref[1].astype(jnp.float32)
                    + bg1_ref[...], 0.0)
    out_ref[...] = jnp.dot(h, w_ref[...], preferred_element_type=jnp.float32
                           ).astype(jnp.bfloat16)


def _final_body(q_ref, bg2_ref, out_ref):
    logits = (q_ref[0].astype(jnp.float32) + q_ref[1].astype(jnp.float32)
              + bg2_ref[...])
    col = lax.broadcasted_iota(jnp.int32, logits.shape, 1)
    valid = col < CLASSES
    masked = jnp.where(valid, logits, -jnp.inf)
    m = jnp.max(masked, axis=1, keepdims=True)
    ex = jnp.where(valid, jnp.exp(logits - m), 0.0)
    lse = jnp.log(jnp.sum(ex, axis=1, keepdims=True)) + m
    res = lse - logits
    out_ref[...] = res[:N_NODES, :CLASSES]


def _lane_bcast(v16, i):
    bidx = jnp.full((16,), i, jnp.int32)
    return lax.gather(
        v16, bidx[:, None],
        dimension_numbers=lax.GatherDimensionNumbers(
            offset_dims=(), collapsed_slice_dims=(0,), start_index_map=(0,)),
        slice_sizes=(1,),
        mode=lax.GatherScatterMode.PROMISE_IN_BOUNDS)


def _slice_starts(d_model):
    starts = list(range(0, d_model - 31, 32))
    if d_model % 32:
        starts.append(d_model - 32)
    return starts


def _make_aggregate(d_model):
    assert d_model % 16 == 0
    mesh = plsc.VectorSubcoreMesh(core_axis_name="c", subcore_axis_name="s",
                                  num_cores=NC, num_subcores=NS)

    @functools.partial(
        pl.kernel,
        out_type=jax.ShapeDtypeStruct((NC, N_PAD, d_model), jnp.bfloat16),
        mesh=mesh,
        compiler_params=pltpu.CompilerParams(use_tc_tiling_on_sc=False,
                                            needs_layout_passes=False),
        scratch_types=[
            pltpu.VMEM((4, CHUNK), jnp.int32),
            pltpu.VMEM((4, CHUNK), jnp.int32),
            pltpu.VMEM((4, CHUNK), jnp.float32),
            pltpu.VMEM((4, CHUNK), jnp.int32),
            pltpu.VMEM((4, CHUNK, d_model), jnp.bfloat16),
            pltpu.VMEM((4, CHUNK, d_model), jnp.bfloat16),
            pltpu.VMEM_SHARED((N_PAD, d_model), jnp.bfloat16),
            pltpu.SemaphoreType.DMA,
            pltpu.SemaphoreType.DMA,
            pltpu.SemaphoreType.DMA,
            pltpu.SemaphoreType.DMA,
            pltpu.SemaphoreType.DMA,
            pltpu.SemaphoreType.DMA,
            pltpu.SemaphoreType.DMA,
            pltpu.SemaphoreType.DMA,
        ],
    )
    def agg(table_hbm, col_hbm, row_hbm, e_hbm, zero_hbm, out_hbm,
            colb, ridxb, eb, sridx, rows_in, rows_out, acc,
            isem0, isem1, gsem0, gsem1, ssem0, ssem1, ssem2, ssem3):
        isem = (isem0, isem1)
        gsem = (gsem0, gsem1)
        ssem = (ssem0, ssem1, ssem2, ssem3)
        cid = lax.axis_index("c")
        sid = lax.axis_index("s")
        wid = cid * NS + sid
        r0 = sid * ROWS_PER_TILE
        base = wid * EDGES_PER_WORKER
        srcs = (col_hbm, row_hbm, e_hbm)
        bufs = (colb, ridxb, eb)

        def coff(c):
            return base + lax.rem(c, NCHUNKS) * CHUNK

        def fire_idx(c, slot, sem):
            for s_hbm, buf in zip(srcs, bufs):
                pltpu.async_copy(s_hbm.at[pl.ds(coff(c), CHUNK)],
                                 buf.at[slot], sem)

        def wait_idx(c, slot, sem):
            for s_hbm, buf in zip(srcs, bufs):
                pltpu.make_async_copy(s_hbm.at[pl.ds(coff(c), CHUNK)],
                                      buf.at[slot], sem).wait()

        def scale(u):
            eb_u = eb.at[u]
            rin = rows_in.at[u]
            rout = rows_out.at[u]
            sr = sridx.at[u]
            rb = ridxb.at[u]
            for g in range(CHUNK // 16):
                sl16 = pl.ds(g * 16, 16)
                sr[sl16] = rb[sl16]
                ev16 = eb_u[sl16]
                for i in range(16):
                    k = g * 16 + i
                    ev = _lane_bcast(ev16, i)
                    evb = plsc.pack(ev, ev, format=plsc.PackFormat.INTERLEAVED)
                    for st in _slice_starts(d_model):
                        sl = pl.ds(st, 32)
                        rout[k, sl] = rin[k, sl] * evb

        def body(j, u, fire=True):
            if fire:
                wait_idx(j + 1, (u + 1) % 4, isem[(u + 1) % 2])
                pltpu.async_copy(table_hbm.at[colb.at[(u + 1) % 4]],
                                 rows_in.at[(u + 1) % 4], gsem[(u + 1) % 2])
            pltpu.make_async_copy(table_hbm.at[colb.at[u]],
                                  rows_in.at[u], gsem[u % 2]).wait()
            pltpu.make_async_copy(table_hbm.at[sridx.at[u]],
                                  rows_out.at[u], ssem[u]).wait()
            scale(u)
            pltpu.async_copy(rows_out.at[u], acc.at[sridx.at[u]],
                             ssem[u], add=True)
            if fire:
                fire_idx(j + 3, (u + 3) % 4, isem[(u + 3) % 2])

        for u in range(4):
            sr = sridx.at[u]
            for g in range(CHUNK // 16):
                sr[pl.ds(g * 16, 16)] = jnp.zeros((16,), jnp.int32)
        fire_idx(0, 0, isem[0])
        wait_idx(0, 0, isem[0])
        fire_idx(1, 1, isem[1])
        fire_idx(2, 2, isem[0])
        pltpu.async_copy(table_hbm.at[colb.at[0]], rows_in.at[0], gsem[0])
        pltpu.sync_copy(zero_hbm.at[pl.ds(r0, ROWS_PER_TILE)],
                        acc.at[pl.ds(r0, ROWS_PER_TILE)])
        plsc.subcore_barrier()
        for u in range(4):
            pltpu.async_copy(table_hbm.at[sridx.at[u]], rows_out.at[u],
                             ssem[u])

        def quad(jj, carry):
            for u in range(4):
                body(4 * jj + u, u)
            return carry

        lax.fori_loop(0, NCHUNKS // 4, quad, 0)
        body(NCHUNKS - 1, 0, fire=False)

        wait_idx(NCHUNKS, 1, isem[1])
        wait_idx(NCHUNKS + 1, 2, isem[0])
        for u in range(4):
            pltpu.make_async_copy(rows_out.at[u], acc.at[sridx.at[u]],
                                  ssem[u]).wait()
        plsc.subcore_barrier()
        pltpu.sync_copy(acc.at[pl.ds(r0, ROWS_PER_TILE)],
                        out_hbm.at[cid, pl.ds(r0, ROWS_PER_TILE)])

    return agg


_agg64 = _make_aggregate(NODE_FILTERS)
_agg48 = _make_aggregate(CLASSES_PAD)


def kernel(x, edges, edge_fs, W1, b1, W2, b2, Wg1, bg1, Wg2, bg2):
    edges = edges.astype(jnp.int32)
    row = edges[0]
    col = edges[1]

    BE = 32000
    NB = N_EDGES // BE
    eft = edge_fs.T
    e2d, xw = pl.pallas_call(
        _edge_mlp_body,
        grid=(NB,),
        in_specs=[
            pl.BlockSpec((D_EDGE, BE), lambda i: (0, i)),
            pl.BlockSpec((EDGE_FILTERS, D_EDGE), lambda i: (0, 0)),
            pl.BlockSpec((EDGE_FILTERS, 1), lambda i: (0, 0)),
            pl.BlockSpec((1, EDGE_FILTERS), lambda i: (0, 0)),
            pl.BlockSpec((1, 1), lambda i: (0, 0)),
            pl.BlockSpec((N_NODES, D_FEAT), lambda i: (0, 0)),
            pl.BlockSpec((D_FEAT, NODE_FILTERS), lambda i: (0, 0)),
        ],
        out_specs=[
            pl.BlockSpec((1, 1, BE), lambda i: (i, 0, 0)),
            pl.BlockSpec((N_PAD, NODE_FILTERS), lambda i: (0, 0)),
        ],
        out_shape=[
            jax.ShapeDtypeStruct((NB, 1, BE), jnp.float32),
            jax.ShapeDtypeStruct((N_PAD, NODE_FILTERS), jnp.bfloat16),
        ],
    )(eft, W1.T, b1.reshape(-1, 1), W2.T, b2.reshape(1, 1), x, Wg1)
    e = e2d.reshape(N_EDGES)

    zeros64 = jnp.zeros((N_PAD, NODE_FILTERS), jnp.bfloat16)
    p1 = _agg64(xw, col, row, e, zeros64)

    Wg2p = jnp.pad(Wg2, ((0, 0), (0, CLASSES_PAD - CLASSES)))
    hw = pl.pallas_call(
        _mid_body,
        out_shape=jax.ShapeDtypeStruct((N_PAD, CLASSES_PAD), jnp.bfloat16),
    )(p1, bg1.reshape(1, -1), Wg2p)

    zeros48 = jnp.zeros((N_PAD, CLASSES_PAD), jnp.bfloat16)
    p2 = _agg48(hw, col, row, e, zeros48)

    bg2p = jnp.pad(bg2, (0, CLASSES_PAD - CLASSES))
    outp = pl.pallas_call(
        _final_body,
        out_shape=jax.ShapeDtypeStruct((N_NODES, CLASSES), jnp.float32),
    )(p2, bg2p.reshape(1, -1))
    return outp
